# baseline (speedup 1.0000x reference)
import functools

import jax
import jax.numpy as jnp
from jax import lax
from jax.experimental import pallas as pl
from jax.experimental.pallas import tpu as pltpu
from jax.experimental.pallas import tpu_sc as plsc

BATCH = 16384
EMB = 64
NUM_NUMERIC = 12
NUM_STYLES = 50

_NC = 2
_NS = 16
_NW = _NC * _NS
_BPW = BATCH // _NW


def _sc_gather(uid2, pid2, utab3, ptab3):
    mesh = plsc.VectorSubcoreMesh(core_axis_name="c", subcore_axis_name="s")

    @functools.partial(
        pl.kernel,
        mesh=mesh,
        out_type=(
            jax.ShapeDtypeStruct((BATCH, EMB), jnp.float32),
            jax.ShapeDtypeStruct((BATCH, EMB), jnp.float32),
        ),
        scratch_types=[
            pltpu.VMEM((2, _BPW), jnp.int32),
            pltpu.SemaphoreType.DMA,
            pltpu.SemaphoreType.DMA,
        ],
    )
    def k(uid_hbm, pid_hbm, utab_hbm, ptab_hbm, uout_hbm, pout_hbm,
          idx_v, usem, psem):
        wid = lax.axis_index("s") * _NC + lax.axis_index("c")
        base = wid * _BPW
        pltpu.sync_copy(uid_hbm.at[wid], idx_v.at[0])
        pltpu.sync_copy(pid_hbm.at[wid], idx_v.at[1])

        def fire(g, _):
            uvec = idx_v[0, pl.ds(g * 16, 16)]
            pvec = idx_v[1, pl.ds(g * 16, 16)]
            for j in range(16):
                ui = uvec[j]
                pi = pvec[j]
                row = base + g * 16 + j
                pltpu.async_copy(utab_hbm.at[ui >> 3, ui & 7],
                                 uout_hbm.at[row], usem)
                pltpu.async_copy(ptab_hbm.at[pi >> 3, pi & 7],
                                 pout_hbm.at[row], psem)
            return 0

        lax.fori_loop(0, _BPW // 16, fire, 0)

        def drain(i, _):
            pltpu.make_async_copy(utab_hbm.at[0, 0],
                                  uout_hbm.at[base + i], usem).wait()
            pltpu.make_async_copy(ptab_hbm.at[0, 0],
                                  pout_hbm.at[base + i], psem).wait()
            return 0

        lax.fori_loop(0, _BPW, drain, 0)

    return k(uid2, pid2, utab3, ptab3)


def _mlp_body(u_ref, p_ref, ff_ref, Wn_ref, bn_ref, Ws_ref, bs_ref,
              W1u_ref, W1p_ref, W1n_ref, W1s_ref, b1_ref,
              W2_ref, b2_ref, W3_ref, b3_ref, wf_ref, bf_ref, o_ref):
    f32 = jnp.float32
    ff = ff_ref[...]
    nvec = jnp.maximum(jnp.dot(ff, Wn_ref[...], preferred_element_type=f32)
                       + bn_ref[...], 0.0)
    svec = jnp.maximum(jnp.dot(ff, Ws_ref[...], preferred_element_type=f32)
                       + bs_ref[...], 0.0)
    h = (jnp.dot(u_ref[...], W1u_ref[...], preferred_element_type=f32)
         + jnp.dot(p_ref[...], W1p_ref[...], preferred_element_type=f32)
         + jnp.dot(nvec, W1n_ref[...], preferred_element_type=f32)
         + jnp.dot(svec, W1s_ref[...], preferred_element_type=f32)
         + b1_ref[...])
    h = jnp.maximum(h, 0.0)
    x2 = jnp.maximum(jnp.dot(h, W2_ref[...], preferred_element_type=f32)
                     + b2_ref[...], 0.0)
    x3 = jnp.maximum(jnp.dot(x2, W3_ref[...], preferred_element_type=f32)
                     + b3_ref[...], 0.0)
    logit = jnp.sum(x3 * wf_ref[...], axis=1, keepdims=True) + bf_ref[...]
    o_ref[...] = jax.nn.sigmoid(logit)


def _mlp(uvec, pvec, ffp, Wn, bn, Ws, bs, W1u, W1p, W1n, W1s, b1,
         W2, b2, W3, b3, wf_row, bf):
    R = 2048
    grid = (BATCH // R,)

    def rows(i):
        return (i, 0)

    def whole(i):
        return (0, 0)

    row_spec = lambda w: pl.BlockSpec((R, w), rows)
    full_spec = lambda a: pl.BlockSpec(a.shape, whole)

    return pl.pallas_call(
        _mlp_body,
        grid=grid,
        in_specs=[
            row_spec(EMB), row_spec(EMB), row_spec(64),
            full_spec(Wn), full_spec(bn), full_spec(Ws), full_spec(bs),
            full_spec(W1u), full_spec(W1p), full_spec(W1n), full_spec(W1s),
            full_spec(b1), full_spec(W2), full_spec(b2),
            full_spec(W3), full_spec(b3), full_spec(wf_row), full_spec(bf),
        ],
        out_specs=pl.BlockSpec((R, 1), rows),
        out_shape=jax.ShapeDtypeStruct((BATCH, 1), jnp.float32),
    )(uvec, pvec, ffp, Wn, bn, Ws, bs, W1u, W1p, W1n, W1s, b1,
      W2, b2, W3, b3, wf_row, bf)


def kernel(user_id, product_id, full_features, user_table, product_table,
           W_num, b_num, W_style, b_style, W1, b1, W2, b2, W3, b3, Wf, bf):
    uid = user_id.astype(jnp.int32)
    pid = product_id.astype(jnp.int32)

    utab3 = user_table.reshape(-1, 8, EMB)
    ptab3 = product_table.reshape(-1, 8, EMB)

    uvec, pvec = _sc_gather(uid.reshape(_NW, _BPW), pid.reshape(_NW, _BPW),
                            utab3, ptab3)

    ffp = jnp.pad(full_features, ((0, 0), (0, 2)))
    Wn = jnp.zeros((64, EMB), jnp.float32).at[:NUM_NUMERIC].set(W_num)
    Ws = jnp.zeros((64, EMB), jnp.float32).at[
        NUM_NUMERIC:NUM_NUMERIC + NUM_STYLES].set(W_style)

    W1u = W1[:EMB]
    W1p = W1[EMB:2 * EMB]
    W1n = W1[2 * EMB:3 * EMB]
    W1s = W1[3 * EMB:]

    return _mlp(uvec, pvec, ffp,
                Wn, b_num.reshape(1, EMB), Ws, b_style.reshape(1, EMB),
                W1u, W1p, W1n, W1s, b1.reshape(1, 128),
                W2, b2.reshape(1, 64), W3, b3.reshape(1, 32),
                Wf.reshape(1, 32), bf.reshape(1, 1))


# speedup vs baseline: 1.0889x; 1.0889x over previous
import functools

import jax
import jax.numpy as jnp
from jax import lax
from jax.experimental import pallas as pl
from jax.experimental.pallas import tpu as pltpu
from jax.experimental.pallas import tpu_sc as plsc

BATCH = 16384
EMB = 64
NUM_NUMERIC = 12
NUM_STYLES = 50

_NC = 2
_NS = 16
_NW = _NC * _NS
_BPW = BATCH // _NW
_CHUNK = 128
_NCHUNK = _BPW // _CHUNK


def _sc_gather(uq, pq, utab2, ptab2):
    mesh = plsc.VectorSubcoreMesh(core_axis_name="c", subcore_axis_name="s")

    @functools.partial(
        pl.kernel,
        mesh=mesh,
        out_type=(
            jax.ShapeDtypeStruct((BATCH, 128), jnp.float32),
            jax.ShapeDtypeStruct((BATCH, 128), jnp.float32),
        ),
        scratch_types=[
            pltpu.VMEM((_NCHUNK, _CHUNK), jnp.int32),
            pltpu.VMEM((_NCHUNK, _CHUNK), jnp.int32),
            pltpu.VMEM((2, _CHUNK, 128), jnp.float32),
            pltpu.VMEM((2, _CHUNK, 128), jnp.float32),
            pltpu.SemaphoreType.DMA,
            pltpu.SemaphoreType.DMA,
        ],
    )
    def k(uq_hbm, pq_hbm, utab_hbm, ptab_hbm, uout_hbm, pout_hbm,
          uq_v, pq_v, ubuf_v, pbuf_v, usem, psem):
        wid = lax.axis_index("s") * _NC + lax.axis_index("c")
        base = wid * _BPW
        pltpu.sync_copy(uq_hbm.at[wid], uq_v)
        pltpu.sync_copy(pq_hbm.at[wid], pq_v)
        cu = [None, None]
        cp = [None, None]
        cu[0] = pltpu.async_copy(utab_hbm.at[uq_v.at[0]], ubuf_v.at[0],
                                 usem)
        cp[0] = pltpu.async_copy(ptab_hbm.at[pq_v.at[0]], pbuf_v.at[0],
                                 psem)
        for j in range(_NCHUNK):
            b = j % 2
            nb = (j + 1) % 2
            if j + 1 < _NCHUNK:
                cu[nb] = pltpu.async_copy(utab_hbm.at[uq_v.at[j + 1]],
                                          ubuf_v.at[nb], usem)
                cp[nb] = pltpu.async_copy(ptab_hbm.at[pq_v.at[j + 1]],
                                          pbuf_v.at[nb], psem)
            cu[b].wait()
            cp[b].wait()
            dst = pl.ds(base + j * _CHUNK, _CHUNK)
            pltpu.sync_copy(ubuf_v.at[b], uout_hbm.at[dst])
            pltpu.sync_copy(pbuf_v.at[b], pout_hbm.at[dst])

    return k(uq, pq, utab2, ptab2)


def _mlp_body(ug_ref, pg_ref, su_ref, sp_ref, ff_ref, Wn_ref, bn_ref,
              Ws_ref, bs_ref, W1u_ref, W1p_ref, W1n_ref, W1s_ref, b1_ref,
              W2_ref, b2_ref, W3_ref, b3_ref, wf_ref, bf_ref, o_ref):
    f32 = jnp.float32
    ug = ug_ref[...]
    pg = pg_ref[...]
    u = jnp.where(su_ref[...] == 1, ug[:, EMB:], ug[:, :EMB])
    p = jnp.where(sp_ref[...] == 1, pg[:, EMB:], pg[:, :EMB])
    ff = ff_ref[...]
    nvec = jnp.maximum(jnp.dot(ff, Wn_ref[...], preferred_element_type=f32)
                       + bn_ref[...], 0.0)
    svec = jnp.maximum(jnp.dot(ff, Ws_ref[...], preferred_element_type=f32)
                       + bs_ref[...], 0.0)
    h = (jnp.dot(u, W1u_ref[...], preferred_element_type=f32)
         + jnp.dot(p, W1p_ref[...], preferred_element_type=f32)
         + jnp.dot(nvec, W1n_ref[...], preferred_element_type=f32)
         + jnp.dot(svec, W1s_ref[...], preferred_element_type=f32)
         + b1_ref[...])
    h = jnp.maximum(h, 0.0)
    x2 = jnp.maximum(jnp.dot(h, W2_ref[...], preferred_element_type=f32)
                     + b2_ref[...], 0.0)
    x3 = jnp.maximum(jnp.dot(x2, W3_ref[...], preferred_element_type=f32)
                     + b3_ref[...], 0.0)
    logit = jnp.sum(x3 * wf_ref[...], axis=1, keepdims=True) + bf_ref[...]
    o_ref[...] = jax.nn.sigmoid(logit)


def _mlp(ug, pg, su, sp, ffp, Wn, bn, Ws, bs, W1u, W1p, W1n, W1s, b1,
         W2, b2, W3, b3, wf_row, bf):
    R = 2048
    grid = (BATCH // R,)

    def rows(i):
        return (i, 0)

    def whole(i):
        return (0, 0)

    row_spec = lambda w: pl.BlockSpec((R, w), rows)
    full_spec = lambda a: pl.BlockSpec(a.shape, whole)

    return pl.pallas_call(
        _mlp_body,
        grid=grid,
        in_specs=[
            row_spec(128), row_spec(128), row_spec(1), row_spec(1),
            row_spec(64),
            full_spec(Wn), full_spec(bn), full_spec(Ws), full_spec(bs),
            full_spec(W1u), full_spec(W1p), full_spec(W1n), full_spec(W1s),
            full_spec(b1), full_spec(W2), full_spec(b2),
            full_spec(W3), full_spec(b3), full_spec(wf_row), full_spec(bf),
        ],
        out_specs=pl.BlockSpec((R, 1), rows),
        out_shape=jax.ShapeDtypeStruct((BATCH, 1), jnp.float32),
    )(ug, pg, su, sp, ffp, Wn, bn, Ws, bs, W1u, W1p, W1n, W1s, b1,
      W2, b2, W3, b3, wf_row, bf)


def kernel(user_id, product_id, full_features, user_table, product_table,
           W_num, b_num, W_style, b_style, W1, b1, W2, b2, W3, b3, Wf, bf):
    uid = user_id.astype(jnp.int32)
    pid = product_id.astype(jnp.int32)

    utab2 = user_table.reshape(-1, 2 * EMB)
    ptab2 = product_table.reshape(-1, 2 * EMB)
    uq = (uid // 2).reshape(_NW, _NCHUNK, _CHUNK)
    pq = (pid // 2).reshape(_NW, _NCHUNK, _CHUNK)
    su = (uid % 2).reshape(BATCH, 1)
    sp = (pid % 2).reshape(BATCH, 1)

    ug, pg = _sc_gather(uq, pq, utab2, ptab2)

    ffp = jnp.pad(full_features, ((0, 0), (0, 2)))
    Wn = jnp.zeros((64, EMB), jnp.float32).at[:NUM_NUMERIC].set(W_num)
    Ws = jnp.zeros((64, EMB), jnp.float32).at[
        NUM_NUMERIC:NUM_NUMERIC + NUM_STYLES].set(W_style)

    W1u = W1[:EMB]
    W1p = W1[EMB:2 * EMB]
    W1n = W1[2 * EMB:3 * EMB]
    W1s = W1[3 * EMB:]

    return _mlp(ug, pg, su, sp, ffp,
                Wn, b_num.reshape(1, EMB), Ws, b_style.reshape(1, EMB),
                W1u, W1p, W1n, W1s, b1.reshape(1, 128),
                W2, b2.reshape(1, 64), W3, b3.reshape(1, 32),
                Wf.reshape(1, 32), bf.reshape(1, 1))


# speedup vs baseline: 2.1250x; 1.9515x over previous
import functools

import jax
import jax.numpy as jnp
from jax import lax
from jax.experimental import pallas as pl
from jax.experimental.pallas import tpu as pltpu
from jax.experimental.pallas import tpu_sc as plsc

BATCH = 16384
EMB = 64
NUM_NUMERIC = 12
NUM_STYLES = 50

_NC = 2
_NS = 16
_NW = _NC * _NS
_BPW = BATCH // _NW
_CHUNK = 128
_NCHUNK = _BPW // _CHUNK

_PACK_R = 4096


def _pack_body(in_ref, o_ref):
    t = jnp.swapaxes(in_ref[...], 0, 1)
    t4 = t.reshape(t.shape[0] // 128, 2, 64, 64)
    left = t4[:, 0].reshape(-1, 64)
    right = t4[:, 1].reshape(-1, 64)
    o_ref[...] = jnp.concatenate([left, right], axis=-1)


def _pack(tabT):
    rows = tabT.shape[1]
    out_rows = 64 * ((rows + 127) // 128)
    grid = (pl.cdiv(out_rows, _PACK_R),)
    return pl.pallas_call(
        _pack_body,
        grid=grid,
        in_specs=[pl.BlockSpec((EMB, 2 * _PACK_R), lambda i: (0, i))],
        out_specs=pl.BlockSpec((_PACK_R, 128), lambda i: (i, 0)),
        out_shape=jax.ShapeDtypeStruct((out_rows, 128), jnp.float32),
    )(tabT)


def _sc_gather(uq, pq, utab2, ptab2):
    mesh = plsc.VectorSubcoreMesh(core_axis_name="c", subcore_axis_name="s")

    @functools.partial(
        pl.kernel,
        mesh=mesh,
        out_type=(
            jax.ShapeDtypeStruct((BATCH, 128), jnp.float32),
            jax.ShapeDtypeStruct((BATCH, 128), jnp.float32),
        ),
        scratch_types=[
            pltpu.VMEM((_NCHUNK, _CHUNK), jnp.int32),
            pltpu.VMEM((_NCHUNK, _CHUNK), jnp.int32),
            pltpu.VMEM((2, _CHUNK, 128), jnp.float32),
            pltpu.VMEM((2, _CHUNK, 128), jnp.float32),
            pltpu.SemaphoreType.DMA,
            pltpu.SemaphoreType.DMA,
        ],
    )
    def k(uq_hbm, pq_hbm, utab_hbm, ptab_hbm, uout_hbm, pout_hbm,
          uq_v, pq_v, ubuf_v, pbuf_v, usem, psem):
        wid = lax.axis_index("s") * _NC + lax.axis_index("c")
        base = wid * _BPW
        pltpu.sync_copy(uq_hbm.at[wid], uq_v)
        pltpu.sync_copy(pq_hbm.at[wid], pq_v)
        cu = [None, None]
        cp = [None, None]
        cu[0] = pltpu.async_copy(utab_hbm.at[uq_v.at[0]], ubuf_v.at[0],
                                 usem)
        cp[0] = pltpu.async_copy(ptab_hbm.at[pq_v.at[0]], pbuf_v.at[0],
                                 psem)
        for j in range(_NCHUNK):
            b = j % 2
            nb = (j + 1) % 2
            if j + 1 < _NCHUNK:
                cu[nb] = pltpu.async_copy(utab_hbm.at[uq_v.at[j + 1]],
                                          ubuf_v.at[nb], usem)
                cp[nb] = pltpu.async_copy(ptab_hbm.at[pq_v.at[j + 1]],
                                          pbuf_v.at[nb], psem)
            cu[b].wait()
            cp[b].wait()
            dst = pl.ds(base + j * _CHUNK, _CHUNK)
            pltpu.sync_copy(ubuf_v.at[b], uout_hbm.at[dst])
            pltpu.sync_copy(pbuf_v.at[b], pout_hbm.at[dst])

    return k(uq, pq, utab2, ptab2)


def _mlp_body(ug_ref, pg_ref, su_ref, sp_ref, ff_ref, Wn_ref, bn_ref,
              Ws_ref, bs_ref, W1u_ref, W1p_ref, W1n_ref, W1s_ref, b1_ref,
              W2_ref, b2_ref, W3_ref, b3_ref, wf_ref, bf_ref, o_ref):
    f32 = jnp.float32
    ug = ug_ref[...]
    pg = pg_ref[...]
    u = jnp.where(su_ref[...] == 1, ug[:, EMB:], ug[:, :EMB])
    p = jnp.where(sp_ref[...] == 1, pg[:, EMB:], pg[:, :EMB])
    ff = ff_ref[...]
    nvec = jnp.maximum(jnp.dot(ff, Wn_ref[...], preferred_element_type=f32)
                       + bn_ref[...], 0.0)
    svec = jnp.maximum(jnp.dot(ff, Ws_ref[...], preferred_element_type=f32)
                       + bs_ref[...], 0.0)
    h = (jnp.dot(u, W1u_ref[...], preferred_element_type=f32)
         + jnp.dot(p, W1p_ref[...], preferred_element_type=f32)
         + jnp.dot(nvec, W1n_ref[...], preferred_element_type=f32)
         + jnp.dot(svec, W1s_ref[...], preferred_element_type=f32)
         + b1_ref[...])
    h = jnp.maximum(h, 0.0)
    x2 = jnp.maximum(jnp.dot(h, W2_ref[...], preferred_element_type=f32)
                     + b2_ref[...], 0.0)
    x3 = jnp.maximum(jnp.dot(x2, W3_ref[...], preferred_element_type=f32)
                     + b3_ref[...], 0.0)
    logit = jnp.sum(x3 * wf_ref[...], axis=1, keepdims=True) + bf_ref[...]
    o_ref[...] = jax.nn.sigmoid(logit)


def _mlp(ug, pg, su, sp, ffp, Wn, bn, Ws, bs, W1u, W1p, W1n, W1s, b1,
         W2, b2, W3, b3, wf_row, bf):
    R = 2048
    grid = (BATCH // R,)

    def rows(i):
        return (i, 0)

    def whole(i):
        return (0, 0)

    row_spec = lambda w: pl.BlockSpec((R, w), rows)
    full_spec = lambda a: pl.BlockSpec(a.shape, whole)

    return pl.pallas_call(
        _mlp_body,
        grid=grid,
        in_specs=[
            row_spec(128), row_spec(128), row_spec(1), row_spec(1),
            row_spec(64),
            full_spec(Wn), full_spec(bn), full_spec(Ws), full_spec(bs),
            full_spec(W1u), full_spec(W1p), full_spec(W1n), full_spec(W1s),
            full_spec(b1), full_spec(W2), full_spec(b2),
            full_spec(W3), full_spec(b3), full_spec(wf_row), full_spec(bf),
        ],
        out_specs=pl.BlockSpec((R, 1), rows),
        out_shape=jax.ShapeDtypeStruct((BATCH, 1), jnp.float32),
    )(ug, pg, su, sp, ffp, Wn, bn, Ws, bs, W1u, W1p, W1n, W1s, b1,
      W2, b2, W3, b3, wf_row, bf)


def kernel(user_id, product_id, full_features, user_table, product_table,
           W_num, b_num, W_style, b_style, W1, b1, W2, b2, W3, b3, Wf, bf):
    uid = user_id.astype(jnp.int32)
    pid = product_id.astype(jnp.int32)

    utab2 = _pack(user_table.T)
    ptab2 = _pack(product_table.T)

    uq = ((uid >> 7) * 64 + (uid & 63)).reshape(_NW, _NCHUNK, _CHUNK)
    pq = ((pid >> 7) * 64 + (pid & 63)).reshape(_NW, _NCHUNK, _CHUNK)
    su = ((uid >> 6) & 1).reshape(BATCH, 1)
    sp = ((pid >> 6) & 1).reshape(BATCH, 1)

    ug, pg = _sc_gather(uq, pq, utab2, ptab2)

    ffp = jnp.pad(full_features, ((0, 0), (0, 2)))
    Wn = jnp.zeros((64, EMB), jnp.float32).at[:NUM_NUMERIC].set(W_num)
    Ws = jnp.zeros((64, EMB), jnp.float32).at[
        NUM_NUMERIC:NUM_NUMERIC + NUM_STYLES].set(W_style)

    W1u = W1[:EMB]
    W1p = W1[EMB:2 * EMB]
    W1n = W1[2 * EMB:3 * EMB]
    W1s = W1[3 * EMB:]

    return _mlp(ug, pg, su, sp, ffp,
                Wn, b_num.reshape(1, EMB), Ws, b_style.reshape(1, EMB),
                W1u, W1p, W1n, W1s, b1.reshape(1, 128),
                W2, b2.reshape(1, 64), W3, b3.reshape(1, 32),
                Wf.reshape(1, 32), bf.reshape(1, 1))


# speedup vs baseline: 2.3375x; 1.1000x over previous
import functools

import jax
import jax.numpy as jnp
from jax import lax
from jax.experimental import pallas as pl
from jax.experimental.pallas import tpu as pltpu
from jax.experimental.pallas import tpu_sc as plsc

BATCH = 16384
EMB = 64
NUM_NUMERIC = 12
NUM_STYLES = 50

_NC = 2
_NS = 16
_NW = _NC * _NS
_BPW = BATCH // _NW
_CHUNK = 128
_NCHUNK = _BPW // _CHUNK

_PACK_R = 8192


def _pack_body(in_ref, o_ref):
    t = jnp.swapaxes(in_ref[...], 0, 1)
    t4 = t.reshape(t.shape[0] // 128, 2, 64, 64)
    left = t4[:, 0].reshape(-1, 64)
    right = t4[:, 1].reshape(-1, 64)
    o_ref[...] = jnp.concatenate([left, right], axis=-1)


def _pack(tabT):
    rows = tabT.shape[1]
    out_rows = 64 * ((rows + 127) // 128)
    grid = (pl.cdiv(out_rows, _PACK_R),)
    return pl.pallas_call(
        _pack_body,
        grid=grid,
        in_specs=[pl.BlockSpec((EMB, 2 * _PACK_R), lambda i: (0, i))],
        out_specs=pl.BlockSpec((_PACK_R, 128), lambda i: (i, 0)),
        out_shape=jax.ShapeDtypeStruct((out_rows, 128), jnp.float32),
    )(tabT)


def _sc_gather(uq, pq, utab2, ptab2):
    mesh = plsc.VectorSubcoreMesh(core_axis_name="c", subcore_axis_name="s")

    @functools.partial(
        pl.kernel,
        mesh=mesh,
        out_type=(
            jax.ShapeDtypeStruct((BATCH, 128), jnp.float32),
            jax.ShapeDtypeStruct((BATCH, 128), jnp.float32),
        ),
        scratch_types=[
            pltpu.VMEM((_NCHUNK, _CHUNK), jnp.int32),
            pltpu.VMEM((_NCHUNK, _CHUNK), jnp.int32),
            pltpu.VMEM((2, _CHUNK, 128), jnp.float32),
            pltpu.VMEM((2, _CHUNK, 128), jnp.float32),
            pltpu.SemaphoreType.DMA,
            pltpu.SemaphoreType.DMA,
        ],
    )
    def k(uq_hbm, pq_hbm, utab_hbm, ptab_hbm, uout_hbm, pout_hbm,
          uq_v, pq_v, ubuf_v, pbuf_v, usem, psem):
        wid = lax.axis_index("s") * _NC + lax.axis_index("c")
        base = wid * _BPW
        pltpu.sync_copy(uq_hbm.at[wid], uq_v)
        pltpu.sync_copy(pq_hbm.at[wid], pq_v)
        cu = [None, None]
        cp = [None, None]
        cu[0] = pltpu.async_copy(utab_hbm.at[uq_v.at[0]], ubuf_v.at[0],
                                 usem)
        cp[0] = pltpu.async_copy(ptab_hbm.at[pq_v.at[0]], pbuf_v.at[0],
                                 psem)
        for j in range(_NCHUNK):
            b = j % 2
            nb = (j + 1) % 2
            if j + 1 < _NCHUNK:
                cu[nb] = pltpu.async_copy(utab_hbm.at[uq_v.at[j + 1]],
                                          ubuf_v.at[nb], usem)
                cp[nb] = pltpu.async_copy(ptab_hbm.at[pq_v.at[j + 1]],
                                          pbuf_v.at[nb], psem)
            cu[b].wait()
            cp[b].wait()
            dst = pl.ds(base + j * _CHUNK, _CHUNK)
            pltpu.sync_copy(ubuf_v.at[b], uout_hbm.at[dst])
            pltpu.sync_copy(pbuf_v.at[b], pout_hbm.at[dst])

    return k(uq, pq, utab2, ptab2)


def _mlp_body(ug_ref, pg_ref, ff_ref, Wn_ref, bn_ref,
              Ws_ref, bs_ref, W1u_ref, W1p_ref, W1n_ref, W1s_ref, b1_ref,
              W2_ref, b2_ref, W3_ref, b3_ref, wf_ref, bf_ref, o_ref):
    f32 = jnp.float32
    ug = ug_ref[...]
    pg = pg_ref[...]
    ff = ff_ref[...]
    u = jnp.where(ff[:, 62:63] > 0.5, ug[:, EMB:], ug[:, :EMB])
    p = jnp.where(ff[:, 63:64] > 0.5, pg[:, EMB:], pg[:, :EMB])
    nvec = jnp.maximum(jnp.dot(ff, Wn_ref[...], preferred_element_type=f32)
                       + bn_ref[...], 0.0)
    svec = jnp.maximum(jnp.dot(ff, Ws_ref[...], preferred_element_type=f32)
                       + bs_ref[...], 0.0)
    h = (jnp.dot(u, W1u_ref[...], preferred_element_type=f32)
         + jnp.dot(p, W1p_ref[...], preferred_element_type=f32)
         + jnp.dot(nvec, W1n_ref[...], preferred_element_type=f32)
         + jnp.dot(svec, W1s_ref[...], preferred_element_type=f32)
         + b1_ref[...])
    h = jnp.maximum(h, 0.0)
    x2 = jnp.maximum(jnp.dot(h, W2_ref[...], preferred_element_type=f32)
                     + b2_ref[...], 0.0)
    x3 = jnp.maximum(jnp.dot(x2, W3_ref[...], preferred_element_type=f32)
                     + b3_ref[...], 0.0)
    logitT = lax.dot_general(wf_ref[...], x3, (((1,), (1,)), ((), ())),
                             preferred_element_type=f32) + bf_ref[...]
    o_ref[...] = jax.nn.sigmoid(logitT)


def _mlp(ug, pg, ffp, Wn, bn, Ws, bs, W1u, W1p, W1n, W1s, b1,
         W2, b2, W3, b3, wf_row, bf):
    R = 4096
    grid = (BATCH // R,)

    def rows(i):
        return (i, 0)

    def whole(i):
        return (0, 0)

    row_spec = lambda w: pl.BlockSpec((R, w), rows)
    full_spec = lambda a: pl.BlockSpec(a.shape, whole)

    return pl.pallas_call(
        _mlp_body,
        grid=grid,
        in_specs=[
            row_spec(128), row_spec(128), row_spec(64),
            full_spec(Wn), full_spec(bn), full_spec(Ws), full_spec(bs),
            full_spec(W1u), full_spec(W1p), full_spec(W1n), full_spec(W1s),
            full_spec(b1), full_spec(W2), full_spec(b2),
            full_spec(W3), full_spec(b3), full_spec(wf_row), full_spec(bf),
        ],
        out_specs=pl.BlockSpec((1, R), lambda i: (0, i)),
        out_shape=jax.ShapeDtypeStruct((1, BATCH), jnp.float32),
    )(ug, pg, ffp, Wn, bn, Ws, bs, W1u, W1p, W1n, W1s, b1,
      W2, b2, W3, b3, wf_row, bf)


def kernel(user_id, product_id, full_features, user_table, product_table,
           W_num, b_num, W_style, b_style, W1, b1, W2, b2, W3, b3, Wf, bf):
    uid = user_id.astype(jnp.int32)
    pid = product_id.astype(jnp.int32)

    utab2 = _pack(user_table.T)
    ptab2 = _pack(product_table.T)

    uq = ((uid >> 7) * 64 + (uid & 63)).reshape(_NW, _NCHUNK, _CHUNK)
    pq = ((pid >> 7) * 64 + (pid & 63)).reshape(_NW, _NCHUNK, _CHUNK)
    su = ((uid >> 6) & 1).astype(jnp.float32).reshape(BATCH, 1)
    sp = ((pid >> 6) & 1).astype(jnp.float32).reshape(BATCH, 1)

    ug, pg = _sc_gather(uq, pq, utab2, ptab2)

    ffp = jnp.concatenate([full_features, su, sp], axis=1)
    Wn = jnp.zeros((64, EMB), jnp.float32).at[:NUM_NUMERIC].set(W_num)
    Ws = jnp.zeros((64, EMB), jnp.float32).at[
        NUM_NUMERIC:NUM_NUMERIC + NUM_STYLES].set(W_style)

    W1u = W1[:EMB]
    W1p = W1[EMB:2 * EMB]
    W1n = W1[2 * EMB:3 * EMB]
    W1s = W1[3 * EMB:]

    res = _mlp(ug, pg, ffp,
               Wn, b_num.reshape(1, EMB), Ws, b_style.reshape(1, EMB),
               W1u, W1p, W1n, W1s, b1.reshape(1, 128),
               W2, b2.reshape(1, 64), W3, b3.reshape(1, 32),
               Wf.reshape(1, 32), bf.reshape(1, 1))
    return res.reshape(BATCH, 1)


# speedup vs baseline: 2.4896x; 1.0650x over previous
import functools

import jax
import jax.numpy as jnp
from jax import lax
from jax.experimental import pallas as pl
from jax.experimental.pallas import tpu as pltpu
from jax.experimental.pallas import tpu_sc as plsc

BATCH = 16384
EMB = 64
NUM_NUMERIC = 12
NUM_STYLES = 50

_NC = 2
_NS = 16
_NW = _NC * _NS
_BPW = BATCH // _NW
_CHUNK = 128
_NCHUNK = _BPW // _CHUNK

_PACK_R = 8192


def _pack_body(in_ref, o_ref):
    x = in_ref[...]
    eye = jnp.eye(EMB, dtype=jnp.float32)
    t = lax.dot_general(x, eye, (((0,), (0,)), ((), ())),
                        preferred_element_type=jnp.float32)
    t4 = t.reshape(t.shape[0] // 128, 2, 64, 64)
    left = t4[:, 0].reshape(-1, 64)
    right = t4[:, 1].reshape(-1, 64)
    o_ref[...] = jnp.concatenate([left, right], axis=-1)


def _pack(tabT):
    rows = tabT.shape[1]
    out_rows = 64 * ((rows + 127) // 128)
    grid = (pl.cdiv(out_rows, _PACK_R),)
    return pl.pallas_call(
        _pack_body,
        grid=grid,
        in_specs=[pl.BlockSpec((EMB, 2 * _PACK_R), lambda i: (0, i))],
        out_specs=pl.BlockSpec((_PACK_R, 128), lambda i: (i, 0)),
        out_shape=jax.ShapeDtypeStruct((out_rows, 128), jnp.float32),
    )(tabT)


def _sc_gather(q3, tab2):
    mesh = plsc.VectorSubcoreMesh(core_axis_name="c", subcore_axis_name="s")

    @functools.partial(
        pl.kernel,
        mesh=mesh,
        out_type=jax.ShapeDtypeStruct((BATCH, 128), jnp.float32),
        scratch_types=[
            pltpu.VMEM((_NCHUNK, _CHUNK), jnp.int32),
            pltpu.VMEM((2, _CHUNK, 128), jnp.float32),
            pltpu.SemaphoreType.DMA,
        ],
    )
    def k(q_hbm, tab_hbm, out_hbm, q_v, buf_v, sem):
        wid = lax.axis_index("s") * _NC + lax.axis_index("c")
        base = wid * _BPW
        pltpu.sync_copy(q_hbm.at[wid], q_v)
        c = [None, None]
        c[0] = pltpu.async_copy(tab_hbm.at[q_v.at[0]], buf_v.at[0], sem)
        for j in range(_NCHUNK):
            b = j % 2
            nb = (j + 1) % 2
            if j + 1 < _NCHUNK:
                c[nb] = pltpu.async_copy(tab_hbm.at[q_v.at[j + 1]],
                                         buf_v.at[nb], sem)
            c[b].wait()
            pltpu.sync_copy(buf_v.at[b],
                            out_hbm.at[pl.ds(base + j * _CHUNK, _CHUNK)])

    return k(q3, tab2)


def _mlp_body(ug_ref, pg_ref, sb_ref, ff_ref, Wn_ref, bn_ref,
              Ws_ref, bs_ref, W1u_ref, W1p_ref, W1n_ref, W1s_ref, b1_ref,
              W2_ref, b2_ref, W3_ref, b3_ref, wf_ref, bf_ref, o_ref):
    f32 = jnp.float32
    ug = ug_ref[...]
    pg = pg_ref[...]
    sb = sb_ref[...]
    u = jnp.where((sb & 1) == 1, ug[:, EMB:], ug[:, :EMB])
    p = jnp.where((sb & 2) == 2, pg[:, EMB:], pg[:, :EMB])
    ff = ff_ref[...]
    nvec = jnp.maximum(jnp.dot(ff, Wn_ref[...], preferred_element_type=f32)
                       + bn_ref[...], 0.0)
    svec = jnp.maximum(jnp.dot(ff, Ws_ref[...], preferred_element_type=f32)
                       + bs_ref[...], 0.0)
    h = (jnp.dot(u, W1u_ref[...], preferred_element_type=f32)
         + jnp.dot(p, W1p_ref[...], preferred_element_type=f32)
         + jnp.dot(nvec, W1n_ref[...], preferred_element_type=f32)
         + jnp.dot(svec, W1s_ref[...], preferred_element_type=f32)
         + b1_ref[...])
    h = jnp.maximum(h, 0.0)
    x2 = jnp.maximum(jnp.dot(h, W2_ref[...], preferred_element_type=f32)
                     + b2_ref[...], 0.0)
    x3 = jnp.maximum(jnp.dot(x2, W3_ref[...], preferred_element_type=f32)
                     + b3_ref[...], 0.0)
    logitT = lax.dot_general(wf_ref[...], x3, (((1,), (1,)), ((), ())),
                             preferred_element_type=f32) + bf_ref[...]
    o_ref[...] = jax.nn.sigmoid(logitT)


def _mlp(ug, pg, sb, ff, Wn, bn, Ws, bs, W1u, W1p, W1n, W1s, b1,
         W2, b2, W3, b3, wf_row, bf):
    R = 4096
    grid = (BATCH // R,)

    def rows(i):
        return (i, 0)

    def whole(i):
        return (0, 0)

    row_spec = lambda w: pl.BlockSpec((R, w), rows)
    full_spec = lambda a: pl.BlockSpec(a.shape, whole)

    return pl.pallas_call(
        _mlp_body,
        grid=grid,
        in_specs=[
            row_spec(128), row_spec(128), row_spec(1), row_spec(62),
            full_spec(Wn), full_spec(bn), full_spec(Ws), full_spec(bs),
            full_spec(W1u), full_spec(W1p), full_spec(W1n), full_spec(W1s),
            full_spec(b1), full_spec(W2), full_spec(b2),
            full_spec(W3), full_spec(b3), full_spec(wf_row), full_spec(bf),
        ],
        out_specs=pl.BlockSpec((1, R), lambda i: (0, i)),
        out_shape=jax.ShapeDtypeStruct((1, BATCH), jnp.float32),
    )(ug, pg, sb, ff, Wn, bn, Ws, bs, W1u, W1p, W1n, W1s, b1,
      W2, b2, W3, b3, wf_row, bf)


def kernel(user_id, product_id, full_features, user_table, product_table,
           W_num, b_num, W_style, b_style, W1, b1, W2, b2, W3, b3, Wf, bf):
    uid = user_id.astype(jnp.int32)
    pid = product_id.astype(jnp.int32)

    uq = ((uid >> 7) * 64 + (uid & 63)).reshape(_NW, _NCHUNK, _CHUNK)
    pq = ((pid >> 7) * 64 + (pid & 63)).reshape(_NW, _NCHUNK, _CHUNK)
    sb = (((uid >> 6) & 1) | (((pid >> 6) & 1) << 1)).reshape(BATCH, 1)

    ptab2 = _pack(product_table.T)
    pg = _sc_gather(pq, ptab2)
    utab2 = _pack(user_table.T)
    ug = _sc_gather(uq, utab2)

    Wn = jnp.zeros((62, EMB), jnp.float32).at[:NUM_NUMERIC].set(W_num)
    Ws = jnp.zeros((62, EMB), jnp.float32).at[NUM_NUMERIC:].set(W_style)

    W1u = W1[:EMB]
    W1p = W1[EMB:2 * EMB]
    W1n = W1[2 * EMB:3 * EMB]
    W1s = W1[3 * EMB:]

    res = _mlp(ug, pg, sb, full_features,
               Wn, b_num.reshape(1, EMB), Ws, b_style.reshape(1, EMB),
               W1u, W1p, W1n, W1s, b1.reshape(1, 128),
               W2, b2.reshape(1, 64), W3, b3.reshape(1, 32),
               Wf.reshape(1, 32), bf.reshape(1, 1))
    return res.reshape(BATCH, 1)


# speedup vs baseline: 2.7359x; 1.0990x over previous
import functools

import jax
import jax.numpy as jnp
from jax import lax
from jax.experimental import pallas as pl
from jax.experimental.pallas import tpu as pltpu
from jax.experimental.pallas import tpu_sc as plsc

BATCH = 16384
EMB = 64
NUM_NUMERIC = 12
NUM_STYLES = 50

_NC = 2
_NS = 16
_NW = _NC * _NS
_BPW = BATCH // _NW
_CHUNK = 128
_NCHUNK = _BPW // _CHUNK

_PACK_R = 16384


def _pack_body(in_ref, o_ref):
    x = in_ref[...]
    eye = jnp.eye(EMB, dtype=jnp.float32)
    t = lax.dot_general(x, eye, (((0,), (0,)), ((), ())),
                        preferred_element_type=jnp.float32)
    t4 = t.reshape(t.shape[0] // 128, 2, 64, 64)
    left = t4[:, 0].reshape(-1, 64)
    right = t4[:, 1].reshape(-1, 64)
    o_ref[...] = jnp.concatenate([left, right], axis=-1)


def _pack(tabT):
    rows = tabT.shape[1]
    out_rows = 64 * ((rows + 127) // 128)
    grid = (pl.cdiv(out_rows, _PACK_R),)
    return pl.pallas_call(
        _pack_body,
        grid=grid,
        in_specs=[pl.BlockSpec((EMB, 2 * _PACK_R), lambda i: (0, i))],
        out_specs=pl.BlockSpec((_PACK_R, 128), lambda i: (i, 0)),
        out_shape=jax.ShapeDtypeStruct((out_rows, 128), jnp.float32),
    )(tabT)


def _sc_gather(q3, tab2):
    mesh = plsc.VectorSubcoreMesh(core_axis_name="c", subcore_axis_name="s")

    @functools.partial(
        pl.kernel,
        mesh=mesh,
        out_type=jax.ShapeDtypeStruct((BATCH, 128), jnp.float32),
        scratch_types=[
            pltpu.VMEM((_NCHUNK, _CHUNK), jnp.int32),
            pltpu.VMEM((2, _CHUNK, 128), jnp.float32),
            pltpu.SemaphoreType.DMA,
        ],
    )
    def k(q_hbm, tab_hbm, out_hbm, q_v, buf_v, sem):
        wid = lax.axis_index("s") * _NC + lax.axis_index("c")
        base = wid * _BPW
        pltpu.sync_copy(q_hbm.at[wid], q_v)
        c = [None, None]
        c[0] = pltpu.async_copy(tab_hbm.at[q_v.at[0]], buf_v.at[0], sem)
        for j in range(_NCHUNK):
            b = j % 2
            nb = (j + 1) % 2
            if j + 1 < _NCHUNK:
                c[nb] = pltpu.async_copy(tab_hbm.at[q_v.at[j + 1]],
                                         buf_v.at[nb], sem)
            c[b].wait()
            pltpu.sync_copy(buf_v.at[b],
                            out_hbm.at[pl.ds(base + j * _CHUNK, _CHUNK)])

    return k(q3, tab2)


def _mlp_body(ug_ref, pg_ref, sb_ref, ffT_ref, Wn_ref, bn_ref,
              Ws_ref, bs_ref, W1_ref, b1_ref,
              W2T_ref, b2_ref, W3T_ref, b3_ref, wf_ref, bf_ref, o_ref):
    f32 = jnp.float32
    dT0 = (((0,), (0,)), ((), ()))
    dT1 = (((1,), (1,)), ((), ()))
    ug = ug_ref[...]
    pg = pg_ref[...]
    sb = sb_ref[...].astype(jnp.int32)
    u = jnp.where((sb & 1) == 1, ug[:, EMB:], ug[:, :EMB])
    p = jnp.where((sb & 2) == 2, pg[:, EMB:], pg[:, :EMB])
    ffT = ffT_ref[...]
    nvec = jnp.maximum(
        lax.dot_general(ffT, Wn_ref[...], dT0, preferred_element_type=f32)
        + bn_ref[...], 0.0)
    svec = jnp.maximum(
        lax.dot_general(ffT, Ws_ref[...], dT0, preferred_element_type=f32)
        + bs_ref[...], 0.0)
    comb = jnp.concatenate([u, p, nvec, svec], axis=-1)
    h = jnp.maximum(
        jnp.dot(comb, W1_ref[...], preferred_element_type=f32)
        + b1_ref[...], 0.0)
    x2 = jnp.maximum(
        lax.dot_general(h, W2T_ref[...], dT1, preferred_element_type=f32)
        + b2_ref[...], 0.0)
    x3 = jnp.maximum(
        lax.dot_general(x2, W3T_ref[...], dT1, preferred_element_type=f32)
        + b3_ref[...], 0.0)
    logitT = lax.dot_general(wf_ref[...], x3, dT1,
                             preferred_element_type=f32) + bf_ref[...]
    o_ref[...] = jax.nn.sigmoid(logitT)


def _mlp(ug, pg, sb, ffT, Wn, bn, Ws, bs, W1, b1,
         W2T, b2, W3T, b3, wf_row, bf):
    R = 4096
    grid = (BATCH // R,)

    def rows(i):
        return (i, 0)

    def whole(i):
        return (0, 0)

    row_spec = lambda w: pl.BlockSpec((R, w), rows)
    full_spec = lambda a: pl.BlockSpec(a.shape, whole)

    return pl.pallas_call(
        _mlp_body,
        grid=grid,
        in_specs=[
            row_spec(128), row_spec(128), row_spec(1),
            pl.BlockSpec((62, R), lambda i: (0, i)),
            full_spec(Wn), full_spec(bn), full_spec(Ws), full_spec(bs),
            full_spec(W1), full_spec(b1),
            full_spec(W2T), full_spec(b2),
            full_spec(W3T), full_spec(b3), full_spec(wf_row), full_spec(bf),
        ],
        out_specs=pl.BlockSpec((1, R), lambda i: (0, i)),
        out_shape=jax.ShapeDtypeStruct((1, BATCH), jnp.float32),
    )(ug, pg, sb, ffT, Wn, bn, Ws, bs, W1, b1,
      W2T, b2, W3T, b3, wf_row, bf)


def kernel(user_id, product_id, full_features, user_table, product_table,
           W_num, b_num, W_style, b_style, W1, b1, W2, b2, W3, b3, Wf, bf):
    uid = user_id.astype(jnp.int32)
    pid = product_id.astype(jnp.int32)

    uq = ((uid >> 7) * 64 + (uid & 63)).reshape(_NW, _NCHUNK, _CHUNK)
    pq = ((pid >> 7) * 64 + (pid & 63)).reshape(_NW, _NCHUNK, _CHUNK)
    sb = (((uid >> 6) & 1) | (((pid >> 6) & 1) << 1)).astype(
        jnp.int8).reshape(BATCH, 1)

    ptab2 = _pack(product_table.T)
    pg = _sc_gather(pq, ptab2)
    utab2 = _pack(user_table.T)
    ug = _sc_gather(uq, utab2)

    Wn = jnp.zeros((62, EMB), jnp.float32).at[:NUM_NUMERIC].set(W_num)
    Ws = jnp.zeros((62, EMB), jnp.float32).at[NUM_NUMERIC:].set(W_style)

    res = _mlp(ug, pg, sb, full_features.T,
               Wn, b_num.reshape(1, EMB), Ws, b_style.reshape(1, EMB),
               W1, b1.reshape(1, 128),
               W2.T, b2.reshape(1, 64), W3.T, b3.reshape(1, 32),
               Wf.reshape(1, 32), bf.reshape(1, 1))
    return res.reshape(BATCH, 1)
